# baseline (speedup 1.0000x reference)
import jax
import jax.numpy as jnp
from jax.experimental import pallas as pl


def _assign_body(x_ref, c_ref, cn_ref, out_ref):
    x4 = x_ref[...]
    xb = jnp.concatenate([x4[:, 32 * g:32 * (g + 1)] for g in range(4)],
                         axis=0)
    s = jnp.dot(xb, c_ref[...] * (-2.0), preferred_element_type=jnp.float32)
    xnorm = jnp.sum(xb * xb, axis=1, keepdims=True)
    dist = (xnorm + s) + cn_ref[...]

    R = dist.shape[0]
    v = dist[:, 0:128]
    j = jnp.zeros((R, 128), jnp.float32)
    for c in (1, 2, 3):
        vc = dist[:, c * 128:(c + 1) * 128]
        jc = jnp.full((R, 128), float(c * 128), jnp.float32)
        take = vc < v
        v = jnp.where(take, vc, v)
        j = jnp.where(take, jc, j)
    lane = jax.lax.broadcasted_iota(jnp.int32, (R, 128), 1).astype(jnp.float32)
    j = j + lane
    vt = v.T
    jt = j.T
    s = 128
    while s > 1:
        h = s // 2
        va, vb = vt[:h], vt[h:s]
        ja, jb = jt[:h], jt[h:s]
        take_b = (vb < va) | ((vb == va) & (jb < ja))
        vt = jnp.where(take_b, vb, va)
        jt = jnp.where(take_b, jb, ja)
        s = h
    out_ref[...] = jt[0].astype(jnp.int32)


def kernel(x, C, Cnorm):
    batched = x.ndim == 3
    x2 = x.reshape(-1, x.shape[-1]) if batched else x
    N, D = x2.shape
    K = C.shape[1]
    R = 2048 if N % 2048 == 0 else N
    xp = x2.reshape(N * D // 128, 128)
    rows = R * D // 128
    out = pl.pallas_call(
        _assign_body,
        grid=(N // R,),
        in_specs=[
            pl.BlockSpec((rows, 128), lambda i: (i, 0)),
            pl.BlockSpec((D, K), lambda i: (0, 0)),
            pl.BlockSpec((1, K), lambda i: (0, 0)),
        ],
        out_specs=pl.BlockSpec((R,), lambda i: (i,)),
        out_shape=jax.ShapeDtypeStruct((N,), jnp.int32),
    )(xp, C, Cnorm)
    out = out.reshape(-1, 4, 512).swapaxes(1, 2).reshape(N)
    return out.reshape(x.shape[:-1]) if batched else out


# speedup vs baseline: 1.7604x; 1.7604x over previous
import jax
import jax.numpy as jnp
from jax.experimental import pallas as pl


def _assign_body(xt_ref, xn_ref, c_ref, cn_ref, out_ref):
    xt = xt_ref[...]
    ct = (c_ref[...] * (-2.0)).T
    s = jnp.dot(ct, xt, preferred_element_type=jnp.float32)
    dist = (xn_ref[...] + s) + cn_ref[...]

    K, R = dist.shape
    nb = K // 8
    v = [dist[8 * b:8 * (b + 1)] for b in range(nb)]
    j = [jnp.full((8, R), float(b), jnp.float32) for b in range(nb)]
    while len(v) > 1:
        nv, nj = [], []
        for a in range(0, len(v), 2):
            take_b = v[a + 1] < v[a]
            nv.append(jnp.where(take_b, v[a + 1], v[a]))
            nj.append(jnp.where(take_b, j[a + 1], j[a]))
        v, j = nv, nj
    vb, jb = v[0], j[0]
    sub = jax.lax.broadcasted_iota(jnp.int32, (8, R), 0).astype(jnp.float32)
    jf = jb * 8.0 + sub
    n = 8
    while n > 1:
        h = n // 2
        va, vbb = vb[:h], vb[h:n]
        ja, jbb = jf[:h], jf[h:n]
        take_b = (vbb < va) | ((vbb == va) & (jbb < ja))
        vb = jnp.where(take_b, vbb, va)
        jf = jnp.where(take_b, jbb, ja)
        n = h
    out_ref[...] = jf[0].astype(jnp.int32)


def kernel(x, C, Cnorm):
    batched = x.ndim == 3
    x2 = x.reshape(-1, x.shape[-1]) if batched else x
    N, D = x2.shape
    K = C.shape[1]
    xt = x2.T
    xn = jnp.sum(x2 ** 2, axis=1, keepdims=True).T
    cnt = Cnorm.T
    R = 2048 if N % 2048 == 0 else N
    out = pl.pallas_call(
        _assign_body,
        grid=(N // R,),
        in_specs=[
            pl.BlockSpec((D, R), lambda i: (0, i)),
            pl.BlockSpec((1, R), lambda i: (0, i)),
            pl.BlockSpec((D, K), lambda i: (0, 0)),
            pl.BlockSpec((K, 1), lambda i: (0, 0)),
        ],
        out_specs=pl.BlockSpec((R,), lambda i: (i,)),
        out_shape=jax.ShapeDtypeStruct((N,), jnp.int32),
    )(xt, xn, C, cnt)
    return out.reshape(x.shape[:-1]) if batched else out


# speedup vs baseline: 1.8493x; 1.0505x over previous
import jax
import jax.numpy as jnp
from jax.experimental import pallas as pl


def _assign_body(xt_ref, xn_ref, c_ref, cn_ref, out_ref):
    xt = xt_ref[...]
    ct = (c_ref[...] * (-2.0)).T
    s = jnp.dot(ct, xt, preferred_element_type=jnp.float32)
    cnt = cn_ref[...].T
    dist = (xn_ref[...] + s) + cnt

    K, R = dist.shape
    nb = K // 8
    v = [dist[8 * b:8 * (b + 1)] for b in range(nb)]
    j = [jnp.full((8, R), float(b), jnp.float32) for b in range(nb)]
    while len(v) > 1:
        nv, nj = [], []
        for a in range(0, len(v), 2):
            take_b = v[a + 1] < v[a]
            nv.append(jnp.where(take_b, v[a + 1], v[a]))
            nj.append(jnp.where(take_b, j[a + 1], j[a]))
        v, j = nv, nj
    vb, jb = v[0], j[0]
    sub = jax.lax.broadcasted_iota(jnp.int32, (8, R), 0).astype(jnp.float32)
    jf = jb * 8.0 + sub
    n = 8
    while n > 1:
        h = n // 2
        va, vbb = vb[:h], vb[h:n]
        ja, jbb = jf[:h], jf[h:n]
        take_b = (vbb < va) | ((vbb == va) & (jbb < ja))
        vb = jnp.where(take_b, vbb, va)
        jf = jnp.where(take_b, jbb, ja)
        n = h
    out_ref[...] = jf[0].astype(jnp.int32)


def kernel(x, C, Cnorm):
    batched = x.ndim == 3
    x2 = x.reshape(-1, x.shape[-1]) if batched else x
    N, D = x2.shape
    K = C.shape[1]
    xt = x2.T
    xn = jnp.sum(x2 ** 2, axis=1, keepdims=True).T
    R = 2048 if N % 2048 == 0 else N
    out = pl.pallas_call(
        _assign_body,
        grid=(N // R,),
        in_specs=[
            pl.BlockSpec((D, R), lambda i: (0, i)),
            pl.BlockSpec((1, R), lambda i: (0, i)),
            pl.BlockSpec((D, K), lambda i: (0, 0)),
            pl.BlockSpec((1, K), lambda i: (0, 0)),
        ],
        out_specs=pl.BlockSpec((R,), lambda i: (i,)),
        out_shape=jax.ShapeDtypeStruct((N,), jnp.int32),
    )(xt, xn, C, Cnorm)
    return out.reshape(x.shape[:-1]) if batched else out


# speedup vs baseline: 2.3045x; 1.2462x over previous
import jax
import jax.numpy as jnp
from jax.experimental import pallas as pl


def _assign_body(xt_ref, xn_ref, c_ref, cn_ref, out_ref):
    xt = xt_ref[...]
    ct = (c_ref[...] * (-2.0)).T
    s = jnp.dot(ct, xt, preferred_element_type=jnp.float32)
    cnt = cn_ref[...].T
    dist = (xn_ref[...] + s) + cnt

    K, R = dist.shape
    nb = K // 8
    v = [dist[8 * b:8 * (b + 1)] for b in range(nb)]
    j = [jnp.full((8, R), float(b), jnp.float32) for b in range(nb)]
    while len(v) > 1:
        nv, nj = [], []
        for a in range(0, len(v), 2):
            take_b = v[a + 1] < v[a]
            nv.append(jnp.where(take_b, v[a + 1], v[a]))
            nj.append(jnp.where(take_b, j[a + 1], j[a]))
        v, j = nv, nj
    vb, jb = v[0], j[0]
    sub = jax.lax.broadcasted_iota(jnp.int32, (8, R), 0).astype(jnp.float32)
    jf = jb * 8.0 + sub
    n = 8
    while n > 1:
        h = n // 2
        va, vbb = vb[:h], vb[h:n]
        ja, jbb = jf[:h], jf[h:n]
        take_b = (vbb < va) | ((vbb == va) & (jbb < ja))
        vb = jnp.where(take_b, vbb, va)
        jf = jnp.where(take_b, jbb, ja)
        n = h
    out_ref[...] = jf[0].astype(jnp.int32).reshape(out_ref.shape)


def kernel(x, C, Cnorm):
    batched = x.ndim == 3
    x2 = x.reshape(-1, x.shape[-1]) if batched else x
    N, D = x2.shape
    K = C.shape[1]
    xt = x2.T
    xn = jnp.sum(x2 ** 2, axis=1, keepdims=True).T
    B = x.shape[0] if batched else 1
    T = N // B
    bb = 8
    R = bb * T
    out = pl.pallas_call(
        _assign_body,
        grid=(N // R,),
        in_specs=[
            pl.BlockSpec((D, R), lambda i: (0, i)),
            pl.BlockSpec((1, R), lambda i: (0, i)),
            pl.BlockSpec((D, K), lambda i: (0, 0)),
            pl.BlockSpec((1, K), lambda i: (0, 0)),
        ],
        out_specs=pl.BlockSpec((bb, T), lambda i: (i, 0)),
        out_shape=jax.ShapeDtypeStruct((B, T), jnp.int32),
    )(xt, xn, C, Cnorm)
    return out if batched else out.reshape(N)


# speedup vs baseline: 2.3229x; 1.0080x over previous
import jax
import jax.numpy as jnp
from jax.experimental import pallas as pl


def _assign_body(xt_ref, xn_ref, c_ref, cn_ref, out_ref):
    xt = xt_ref[...]
    ct = (c_ref[...] * (-2.0)).T
    s = jnp.dot(ct, xt, preferred_element_type=jnp.float32)
    cnt = cn_ref[...].T
    dist = (xn_ref[...] + s) + cnt

    K, R = dist.shape
    nb = K // 8
    v = [dist[8 * b:8 * (b + 1)] for b in range(nb)]
    j = [jnp.full((8, R), float(b), jnp.float32) for b in range(nb)]
    while len(v) > 1:
        nv, nj = [], []
        for a in range(0, len(v), 2):
            take_b = v[a + 1] < v[a]
            nv.append(jnp.where(take_b, v[a + 1], v[a]))
            nj.append(jnp.where(take_b, j[a + 1], j[a]))
        v, j = nv, nj
    vb, jb = v[0], j[0]
    sub = jax.lax.broadcasted_iota(jnp.int32, (8, R), 0).astype(jnp.float32)
    jf = jb * 8.0 + sub
    n = 8
    while n > 1:
        h = n // 2
        va, vbb = vb[:h], vb[h:n]
        ja, jbb = jf[:h], jf[h:n]
        take_b = (vbb < va) | ((vbb == va) & (jbb < ja))
        vb = jnp.where(take_b, vbb, va)
        jf = jnp.where(take_b, jbb, ja)
        n = h
    out_ref[...] = jf[0].astype(jnp.int32).reshape(out_ref.shape)


def kernel(x, C, Cnorm):
    batched = x.ndim == 3
    x2 = x.reshape(-1, x.shape[-1]) if batched else x
    N, D = x2.shape
    K = C.shape[1]
    xt = x2.T
    xn = jnp.sum(x2 ** 2, axis=1).reshape(1, N)
    B = x.shape[0] if batched else 1
    T = N // B
    bb = 8
    R = bb * T
    out = pl.pallas_call(
        _assign_body,
        grid=(N // R,),
        in_specs=[
            pl.BlockSpec((D, R), lambda i: (0, i)),
            pl.BlockSpec((1, R), lambda i: (0, i)),
            pl.BlockSpec((D, K), lambda i: (0, 0)),
            pl.BlockSpec((1, K), lambda i: (0, 0)),
        ],
        out_specs=pl.BlockSpec((bb, T), lambda i: (i, 0)),
        out_shape=jax.ShapeDtypeStruct((B, T), jnp.int32),
    )(xt, xn, C, Cnorm)
    return out if batched else out.reshape(N)
